# Initial kernel scaffold; baseline (speedup 1.0000x reference)
import jax
import jax.numpy as jnp
from jax.experimental import pallas as pl


def kernel(x, emb):
    raise NotImplementedError("write your pallas kernel here")


# speedup vs baseline: 1.3979x; 1.3979x over previous
import functools

import jax
import jax.numpy as jnp
from jax import lax
from jax.experimental import pallas as pl
from jax.experimental.pallas import tpu as pltpu
from jax.experimental.pallas import tpu_sc as plsc

_INFO = plsc.get_sparse_core_info()
_NC = _INFO.num_cores
_NS = _INFO.num_subcores
_L = _INFO.num_lanes
_NW = _NC * _NS


def _make_sc_kernel(seq_len, dim, scale):
    rows_per_w = seq_len // _NW
    ch = 32
    nch = rows_per_w // ch
    nbuf = 3

    mesh = plsc.VectorSubcoreMesh(core_axis_name="c", subcore_axis_name="s")

    @functools.partial(
        pl.kernel,
        out_type=jax.ShapeDtypeStruct((seq_len, dim), jnp.float32),
        mesh=mesh,
        scratch_types=[
            pltpu.VMEM((nbuf, ch, dim), jnp.float32),
            pltpu.SemaphoreType.DMA((nbuf,)),
            pltpu.SemaphoreType.DMA((nbuf,)),
        ],
    )
    def k(emb_hbm, out_hbm, buf, in_sems, out_sems):
        wid = lax.axis_index("s") * _NC + lax.axis_index("c")
        base = wid * rows_per_w

        def in_copy(g):
            b = g % nbuf
            return pltpu.make_async_copy(
                emb_hbm.at[pl.ds(base + g * ch, ch)], buf.at[b], in_sems.at[b])

        def out_copy(g):
            b = g % nbuf
            return pltpu.make_async_copy(
                buf.at[b], out_hbm.at[pl.ds(base + g * ch, ch)], out_sems.at[b])

        in_copy(0).start()
        if nch > 1:
            in_copy(1).start()
        for g in range(nch):
            b = g % nbuf
            if g + 2 < nch:
                if g >= 1:
                    out_copy(g - 1).wait()
                in_copy(g + 2).start()
            in_copy(g).wait()

            def row_body(r, _):
                for c in range(dim // _L):
                    sl = pl.ds(c * _L, _L)
                    buf[b, r, sl] = buf[b, r, sl] * scale
                return 0

            lax.fori_loop(0, ch, row_body, 0, unroll=False)
            out_copy(g).start()
        for g in range(max(0, nch - nbuf), nch):
            out_copy(g).wait()

    return k


def kernel(x, emb):
    seq_len = x.shape[1]
    dim = emb.shape[1]
    scale = dim ** (-0.5)
    return _make_sc_kernel(seq_len, dim, scale)(emb[:seq_len])


# speedup vs baseline: 2.7443x; 1.9632x over previous
import functools

import jax
import jax.numpy as jnp
from jax import lax
from jax.experimental import pallas as pl
from jax.experimental.pallas import tpu as pltpu
from jax.experimental.pallas import tpu_sc as plsc

_INFO = plsc.get_sparse_core_info()
_NC = _INFO.num_cores
_NS = _INFO.num_subcores
_L = _INFO.num_lanes
_NW = _NC * _NS


def _make_sc_kernel(seq_len, dim, scale):
    rows_per_w = seq_len // _NW
    ch = 32
    nch = rows_per_w // ch
    nbuf = 3

    mesh = plsc.VectorSubcoreMesh(core_axis_name="c", subcore_axis_name="s")

    @functools.partial(
        pl.kernel,
        out_type=jax.ShapeDtypeStruct((seq_len, dim), jnp.float32),
        mesh=mesh,
        scratch_types=[
            pltpu.VMEM((nbuf, ch, dim), jnp.float32),
            pltpu.SemaphoreType.DMA((nbuf,)),
            pltpu.SemaphoreType.DMA((nbuf,)),
        ],
    )
    def k(emb_hbm, out_hbm, buf, in_sems, out_sems):
        wid = lax.axis_index("s") * _NC + lax.axis_index("c")
        base = wid * rows_per_w

        def in_copy(g):
            b = g % nbuf
            return pltpu.make_async_copy(
                emb_hbm.at[pl.ds(base + g * ch, ch)], buf.at[b], in_sems.at[b])

        def out_copy(g):
            b = g % nbuf
            return pltpu.make_async_copy(
                buf.at[b], out_hbm.at[pl.ds(base + g * ch, ch)], out_sems.at[b])

        in_copy(0).start()
        if nch > 1:
            in_copy(1).start()
        for g in range(nch):
            b = g % nbuf
            if g + 2 < nch:
                if g >= 1:
                    out_copy(g - 1).wait()
                in_copy(g + 2).start()
            in_copy(g).wait()

            def row_body(r, _):
                for c in range(dim // _L):
                    sl = pl.ds(c * _L, _L)
                    buf[b, r, sl] = buf[b, r, sl] * scale
                return 0

            lax.fori_loop(0, ch, row_body, 0, unroll=False)
            out_copy(g).start()
        for g in range(max(0, nch - nbuf), nch):
            out_copy(g).wait()

    return k


def _tc_scale_copy(e, rows_per_block, scale):
    seq_len, dim = e.shape

    def body(e_ref, o_ref):
        o_ref[...] = e_ref[...] * scale

    return pl.pallas_call(
        body,
        grid=(seq_len // rows_per_block,),
        in_specs=[pl.BlockSpec((rows_per_block, dim), lambda i: (i, 0))],
        out_specs=pl.BlockSpec((rows_per_block, dim), lambda i: (i, 0)),
        out_shape=jax.ShapeDtypeStruct((seq_len, dim), e.dtype),
    )(e)


def kernel(x, emb):
    seq_len = x.shape[1]
    dim = emb.shape[1]
    scale = dim ** (-0.5)
    return _tc_scale_copy(emb[:seq_len], 512, scale)


# speedup vs baseline: 3.0092x; 1.0965x over previous
import functools

import jax
import jax.numpy as jnp
from jax import lax
from jax.experimental import pallas as pl
from jax.experimental.pallas import tpu as pltpu
from jax.experimental.pallas import tpu_sc as plsc

_INFO = plsc.get_sparse_core_info()
_NC = _INFO.num_cores
_NS = _INFO.num_subcores
_L = _INFO.num_lanes
_NW = _NC * _NS


def _make_sc_kernel(seq_len, dim, scale):
    rows_per_w = seq_len // _NW
    ch = 32
    nch = rows_per_w // ch
    nbuf = 3

    mesh = plsc.VectorSubcoreMesh(core_axis_name="c", subcore_axis_name="s")

    @functools.partial(
        pl.kernel,
        out_type=jax.ShapeDtypeStruct((seq_len, dim), jnp.float32),
        mesh=mesh,
        scratch_types=[
            pltpu.VMEM((nbuf, ch, dim), jnp.float32),
            pltpu.SemaphoreType.DMA((nbuf,)),
            pltpu.SemaphoreType.DMA((nbuf,)),
        ],
    )
    def k(emb_hbm, out_hbm, buf, in_sems, out_sems):
        wid = lax.axis_index("s") * _NC + lax.axis_index("c")
        base = wid * rows_per_w

        def in_copy(g):
            b = g % nbuf
            return pltpu.make_async_copy(
                emb_hbm.at[pl.ds(base + g * ch, ch)], buf.at[b], in_sems.at[b])

        def out_copy(g):
            b = g % nbuf
            return pltpu.make_async_copy(
                buf.at[b], out_hbm.at[pl.ds(base + g * ch, ch)], out_sems.at[b])

        in_copy(0).start()
        if nch > 1:
            in_copy(1).start()
        for g in range(nch):
            b = g % nbuf
            if g + 2 < nch:
                if g >= 1:
                    out_copy(g - 1).wait()
                in_copy(g + 2).start()
            in_copy(g).wait()

            def row_body(r, _):
                for c in range(dim // _L):
                    sl = pl.ds(c * _L, _L)
                    buf[b, r, sl] = buf[b, r, sl] * scale
                return 0

            lax.fori_loop(0, ch, row_body, 0, unroll=False)
            out_copy(g).start()
        for g in range(max(0, nch - nbuf), nch):
            out_copy(g).wait()

    return k


def _tc_scale_copy(e, rows_per_block, scale):
    seq_len, dim = e.shape

    def body(e_ref, o_ref):
        o_ref[...] = e_ref[...] * scale

    return pl.pallas_call(
        body,
        grid=(seq_len // rows_per_block,),
        in_specs=[pl.BlockSpec((rows_per_block, dim), lambda i: (i, 0))],
        out_specs=pl.BlockSpec((rows_per_block, dim), lambda i: (i, 0)),
        out_shape=jax.ShapeDtypeStruct((seq_len, dim), e.dtype),
    )(e)


def kernel(x, emb):
    seq_len = x.shape[1]
    dim = emb.shape[1]
    scale = dim ** (-0.5)
    return _tc_scale_copy(emb[:seq_len], 1024, scale)


# speedup vs baseline: 3.2363x; 1.0755x over previous
import functools

import jax
import jax.numpy as jnp
from jax import lax
from jax.experimental import pallas as pl
from jax.experimental.pallas import tpu as pltpu
from jax.experimental.pallas import tpu_sc as plsc

_INFO = plsc.get_sparse_core_info()
_NC = _INFO.num_cores
_NS = _INFO.num_subcores
_L = _INFO.num_lanes
_NW = _NC * _NS


def _make_sc_kernel(seq_len, dim, scale):
    rows_per_w = seq_len // _NW
    ch = 32
    nch = rows_per_w // ch
    nbuf = 3

    mesh = plsc.VectorSubcoreMesh(core_axis_name="c", subcore_axis_name="s")

    @functools.partial(
        pl.kernel,
        out_type=jax.ShapeDtypeStruct((seq_len, dim), jnp.float32),
        mesh=mesh,
        scratch_types=[
            pltpu.VMEM((nbuf, ch, dim), jnp.float32),
            pltpu.SemaphoreType.DMA((nbuf,)),
            pltpu.SemaphoreType.DMA((nbuf,)),
        ],
    )
    def k(emb_hbm, out_hbm, buf, in_sems, out_sems):
        wid = lax.axis_index("s") * _NC + lax.axis_index("c")
        base = wid * rows_per_w

        def in_copy(g):
            b = g % nbuf
            return pltpu.make_async_copy(
                emb_hbm.at[pl.ds(base + g * ch, ch)], buf.at[b], in_sems.at[b])

        def out_copy(g):
            b = g % nbuf
            return pltpu.make_async_copy(
                buf.at[b], out_hbm.at[pl.ds(base + g * ch, ch)], out_sems.at[b])

        in_copy(0).start()
        if nch > 1:
            in_copy(1).start()
        for g in range(nch):
            b = g % nbuf
            if g + 2 < nch:
                if g >= 1:
                    out_copy(g - 1).wait()
                in_copy(g + 2).start()
            in_copy(g).wait()

            def row_body(r, _):
                for c in range(dim // _L):
                    sl = pl.ds(c * _L, _L)
                    buf[b, r, sl] = buf[b, r, sl] * scale
                return 0

            lax.fori_loop(0, ch, row_body, 0, unroll=False)
            out_copy(g).start()
        for g in range(max(0, nch - nbuf), nch):
            out_copy(g).wait()

    return k


def _tc_scale_copy(e, rows_per_block, scale):
    seq_len, dim = e.shape

    def body(e_ref, o_ref):
        o_ref[...] = e_ref[...] * scale

    return pl.pallas_call(
        body,
        grid=(seq_len // rows_per_block,),
        in_specs=[pl.BlockSpec((rows_per_block, dim), lambda i: (i, 0))],
        out_specs=pl.BlockSpec((rows_per_block, dim), lambda i: (i, 0)),
        out_shape=jax.ShapeDtypeStruct((seq_len, dim), e.dtype),
    )(e)


def kernel(x, emb):
    seq_len = x.shape[1]
    dim = emb.shape[1]
    scale = dim ** (-0.5)
    return _tc_scale_copy(emb[:seq_len], 2048, scale)
